# initial kernel scaffold (unmeasured)
import jax
import jax.numpy as jnp
from jax import lax
from jax.experimental import pallas as pl
from jax.experimental.pallas import tpu as pltpu


def kernel(
    x,
):
    def body(*refs):
        pass

    out_shape = jax.ShapeDtypeStruct(..., jnp.float32)
    return pl.pallas_call(body, out_shape=out_shape)(...)


# baseline (device time: 97713 ns/iter reference)
import jax
import jax.numpy as jnp
from jax import lax
from jax.experimental import pallas as pl
from jax.experimental.pallas import tpu as pltpu

N_DEV = 8
K = 32
POOL_B = 32
POOL_J = 8
COL_BLK = 256


def _local_topk_t(xt):
    n, m = xt.shape
    blk_l = n // POOL_B

    def body(x_ref, out_ref):
        xv = x_ref[...].reshape(POOL_B, blk_l, COL_BLK)
        pool = []
        for _ in range(POOL_J):
            v = jnp.max(xv, axis=1)
            xv = jnp.where(xv >= v[:, None, :], float("-inf"), xv)
            pool.append(v)
        p = jnp.concatenate(pool, axis=0)
        vals = []
        for _ in range(K):
            v = jnp.max(p, axis=0)
            p = jnp.where(p >= v[None, :], float("-inf"), p)
            vals.append(v)
        out_ref[...] = jnp.stack(vals, axis=0)

    return pl.pallas_call(
        body,
        grid=(m // COL_BLK,),
        in_specs=[pl.BlockSpec((n, COL_BLK), lambda i: (0, i))],
        out_specs=pl.BlockSpec((K, COL_BLK), lambda i: (0, i)),
        out_shape=jax.ShapeDtypeStruct((K, m), jnp.float32),
    )(xt)


def _allgather_merge_t(cand):
    k, m = cand.shape

    def body(c_ref, out_ref, gath_ref, send_sems, recv_sems):
        my = lax.axis_index("i")
        left = jnp.mod(my - 1, N_DEV)
        right = jnp.mod(my + 1, N_DEV)

        barrier_sem = pltpu.get_barrier_semaphore()
        for nbr in [left, right]:
            pl.semaphore_signal(
                barrier_sem, inc=1,
                device_id=(nbr,), device_id_type=pl.DeviceIdType.MESH,
            )
        pl.semaphore_wait(barrier_sem, 2)

        gath_ref[pl.ds(my, 1), :, :] = c_ref[...][None, :, :]

        for h in range(N_DEV - 1):
            slot = jnp.mod(my - h, N_DEV)
            rdma = pltpu.make_async_remote_copy(
                src_ref=gath_ref.at[slot],
                dst_ref=gath_ref.at[slot],
                send_sem=send_sems.at[h],
                recv_sem=recv_sems.at[h],
                device_id=(right,),
                device_id_type=pl.DeviceIdType.MESH,
            )
            rdma.start()
            rdma.wait()

        p = gath_ref[...].reshape(N_DEV * k, m)
        vals = []
        for _ in range(k):
            v = jnp.max(p, axis=0)
            p = jnp.where(p >= v[None, :], float("-inf"), p)
            vals.append(v)
        out_ref[...] = jnp.stack(vals, axis=0)

    return pl.pallas_call(
        body,
        out_shape=jax.ShapeDtypeStruct((k, m), jnp.float32),
        in_specs=[pl.BlockSpec(memory_space=pltpu.VMEM)],
        out_specs=pl.BlockSpec(memory_space=pltpu.VMEM),
        scratch_shapes=[
            pltpu.VMEM((N_DEV, k, m), jnp.float32),
            pltpu.SemaphoreType.DMA((N_DEV - 1,)),
            pltpu.SemaphoreType.DMA((N_DEV - 1,)),
        ],
        compiler_params=pltpu.CompilerParams(collective_id=0),
    )(cand)


def kernel(x):
    xt = x.T
    cand = _local_topk_t(xt)
    out_t = _allgather_merge_t(cand)
    return out_t.T
